# initial kernel scaffold (unmeasured)
import jax
import jax.numpy as jnp
from jax import lax
from jax.experimental import pallas as pl
from jax.experimental.pallas import tpu as pltpu


def kernel(
    x,
):
    def body(*refs):
        pass

    out_shape = jax.ShapeDtypeStruct(..., jnp.float32)
    return pl.pallas_call(body, out_shape=out_shape)(...)


# baseline (device time: 30677 ns/iter reference)
import jax
import jax.numpy as jnp
from jax import lax
from jax.experimental import pallas as pl
from jax.experimental.pallas import tpu as pltpu

N_DEV = 4


def kernel(x):
    x2 = x.reshape(x.shape[1], x.shape[2])
    m, n = x2.shape

    def body(x_ref, out_ref, comm_ref, send_sems, recv_sems):
        my = lax.axis_index("i")
        p1 = my ^ 1
        p2 = 3 - my

        barrier_sem = pltpu.get_barrier_semaphore()
        for nbr in [p1, p2]:
            pl.semaphore_signal(
                barrier_sem, inc=1,
                device_id=(nbr,), device_id_type=pl.DeviceIdType.MESH,
            )
        pl.semaphore_wait(barrier_sem, 2)

        rdma1 = pltpu.make_async_remote_copy(
            src_ref=x_ref,
            dst_ref=comm_ref.at[0],
            send_sem=send_sems.at[0],
            recv_sem=recv_sems.at[0],
            device_id=(p1,),
            device_id_type=pl.DeviceIdType.MESH,
        )
        rdma1.start()
        rdma1.wait()
        out_ref[...] = x_ref[...] + comm_ref[0]

        rdma2 = pltpu.make_async_remote_copy(
            src_ref=out_ref,
            dst_ref=comm_ref.at[1],
            send_sem=send_sems.at[1],
            recv_sem=recv_sems.at[1],
            device_id=(p2,),
            device_id_type=pl.DeviceIdType.MESH,
        )
        rdma2.start()
        rdma2.wait()
        out_ref[...] = out_ref[...] + comm_ref[1]

    return pl.pallas_call(
        body,
        out_shape=jax.ShapeDtypeStruct((m, n), x2.dtype),
        in_specs=[pl.BlockSpec(memory_space=pltpu.VMEM)],
        out_specs=pl.BlockSpec(memory_space=pltpu.VMEM),
        scratch_shapes=[
            pltpu.VMEM((2, m, n), x2.dtype),
            pltpu.SemaphoreType.DMA((2,)),
            pltpu.SemaphoreType.DMA((2,)),
        ],
        compiler_params=pltpu.CompilerParams(collective_id=0),
    )(x2)


# device time: 19837 ns/iter; 1.5465x vs baseline; 1.5465x over previous
import jax
import jax.numpy as jnp
from jax import lax
from jax.experimental import pallas as pl
from jax.experimental.pallas import tpu as pltpu

N_DEV = 4


def kernel(x):
    x2 = x.reshape(x.shape[1], x.shape[2])
    m, n = x2.shape
    h = m // 2

    def body(x_ref, out_ref, comm_ref, send_sems, recv_sems):
        my = lax.axis_index("i")
        p1 = my ^ 1
        p2 = 3 - my

        barrier_sem = pltpu.get_barrier_semaphore()
        for nbr in [p1, p2]:
            pl.semaphore_signal(
                barrier_sem, inc=1,
                device_id=(nbr,), device_id_type=pl.DeviceIdType.MESH,
            )
        pl.semaphore_wait(barrier_sem, 2)

        r1a = pltpu.make_async_remote_copy(
            src_ref=x_ref.at[pl.ds(0, h), :],
            dst_ref=comm_ref.at[0],
            send_sem=send_sems.at[0],
            recv_sem=recv_sems.at[0],
            device_id=(p1,),
            device_id_type=pl.DeviceIdType.MESH,
        )
        r1b = pltpu.make_async_remote_copy(
            src_ref=x_ref.at[pl.ds(h, h), :],
            dst_ref=comm_ref.at[1],
            send_sem=send_sems.at[1],
            recv_sem=recv_sems.at[1],
            device_id=(p2,),
            device_id_type=pl.DeviceIdType.MESH,
        )
        r1a.start()
        r1b.start()
        r1a.wait_recv()
        out_ref[pl.ds(0, h), :] = x_ref[pl.ds(0, h), :] + comm_ref[0]
        r1b.wait_recv()
        out_ref[pl.ds(h, h), :] = x_ref[pl.ds(h, h), :] + comm_ref[1]

        r2a = pltpu.make_async_remote_copy(
            src_ref=out_ref.at[pl.ds(0, h), :],
            dst_ref=comm_ref.at[2],
            send_sem=send_sems.at[2],
            recv_sem=recv_sems.at[2],
            device_id=(p2,),
            device_id_type=pl.DeviceIdType.MESH,
        )
        r2b = pltpu.make_async_remote_copy(
            src_ref=out_ref.at[pl.ds(h, h), :],
            dst_ref=comm_ref.at[3],
            send_sem=send_sems.at[3],
            recv_sem=recv_sems.at[3],
            device_id=(p1,),
            device_id_type=pl.DeviceIdType.MESH,
        )
        r2a.start()
        r2b.start()
        r1a.wait_send()
        r1b.wait_send()
        r2a.wait()
        out_ref[pl.ds(0, h), :] = out_ref[pl.ds(0, h), :] + comm_ref[2]
        r2b.wait()
        out_ref[pl.ds(h, h), :] = out_ref[pl.ds(h, h), :] + comm_ref[3]

    return pl.pallas_call(
        body,
        out_shape=jax.ShapeDtypeStruct((m, n), x2.dtype),
        in_specs=[pl.BlockSpec(memory_space=pltpu.VMEM)],
        out_specs=pl.BlockSpec(memory_space=pltpu.VMEM),
        scratch_shapes=[
            pltpu.VMEM((4, h, n), x2.dtype),
            pltpu.SemaphoreType.DMA((4,)),
            pltpu.SemaphoreType.DMA((4,)),
        ],
        compiler_params=pltpu.CompilerParams(collective_id=0),
    )(x2)


# device time: 18509 ns/iter; 1.6574x vs baseline; 1.0717x over previous
import jax
import jax.numpy as jnp
from jax import lax
from jax.experimental import pallas as pl
from jax.experimental.pallas import tpu as pltpu

N_DEV = 4
N_CHUNK = 2


def kernel(x):
    x2 = x.reshape(x.shape[1], x.shape[2])
    m, n = x2.shape
    q = m // (2 * N_CHUNK)

    def body(x_ref, out_ref, comm_ref, send_sems, recv_sems):
        my = lax.axis_index("i")
        p1 = my ^ 1
        p2 = 3 - my

        barrier_sem = pltpu.get_barrier_semaphore()
        for nbr in [p1, p2]:
            pl.semaphore_signal(
                barrier_sem, inc=1,
                device_id=(nbr,), device_id_type=pl.DeviceIdType.MESH,
            )
        pl.semaphore_wait(barrier_sem, 2)

        n_half = 2 * N_CHUNK

        def mk(src, slot, dst_dev):
            return pltpu.make_async_remote_copy(
                src_ref=src,
                dst_ref=comm_ref.at[slot],
                send_sem=send_sems.at[slot],
                recv_sem=recv_sems.at[slot],
                device_id=(dst_dev,),
                device_id_type=pl.DeviceIdType.MESH,
            )

        r1 = []
        for c in range(n_half):
            dev = p1 if c < N_CHUNK else p2
            r1.append(mk(x_ref.at[pl.ds(c * q, q), :], c, dev))
        for rdma in r1:
            rdma.start()

        order = [c for pair in zip(range(N_CHUNK), range(N_CHUNK, n_half))
                 for c in pair]
        r2 = [None] * n_half
        for c in order:
            dev = p2 if c < N_CHUNK else p1
            r1[c].wait_recv()
            rows = pl.ds(c * q, q)
            out_ref[rows, :] = x_ref[rows, :] + comm_ref[c]
            r2[c] = mk(out_ref.at[rows, :], n_half + c, dev)
            r2[c].start()

        for c in order:
            r2[c].wait()
            rows = pl.ds(c * q, q)
            out_ref[rows, :] = out_ref[rows, :] + comm_ref[n_half + c]

        for rdma in r1:
            rdma.wait_send()

    return pl.pallas_call(
        body,
        out_shape=jax.ShapeDtypeStruct((m, n), x2.dtype),
        in_specs=[pl.BlockSpec(memory_space=pltpu.VMEM)],
        out_specs=pl.BlockSpec(memory_space=pltpu.VMEM),
        scratch_shapes=[
            pltpu.VMEM((4 * N_CHUNK, q, n), x2.dtype),
            pltpu.SemaphoreType.DMA((4 * N_CHUNK,)),
            pltpu.SemaphoreType.DMA((4 * N_CHUNK,)),
        ],
        compiler_params=pltpu.CompilerParams(collective_id=0),
    )(x2)
